# initial kernel scaffold (unmeasured)
import jax
import jax.numpy as jnp
from jax import lax
from jax.experimental import pallas as pl
from jax.experimental.pallas import tpu as pltpu

B, S, D, DC, H, Dh, Dr = 2, 512, 2048, 256, 16, 128, 32
DC_SH = DC // 2
BS = B * S
SCALE = (Dh + Dr) ** -0.5


def kernel(x, Wdkv, Wuk, Wuv, Wq, Wqr, Wkr, Wo):
    def body(x_ref, wdkv_ref, wuk_ref, wuv_ref, wq_ref, wqr_ref, wkr_ref,
             wo_ref, out_ref,
             c_loc, c_nbr, wuk_nbr, wuv_nbr,
             q_buf, qr_buf, kr_buf, k_buf, v_buf, o_buf,
             send_sems, recv_sems):
        my_x = lax.axis_index("x")
        my_y = lax.axis_index("y")
        nbr = (1 - my_x, my_y)

        barrier = pltpu.get_barrier_semaphore()
        pl.semaphore_signal(barrier, inc=1, device_id=nbr,
                            device_id_type=pl.DeviceIdType.MESH)
        pl.semaphore_wait(barrier, 1)

        for b in range(B):
            c_loc[pl.ds(b * S, S), :] = jnp.dot(
                x_ref[b], wdkv_ref[:], preferred_element_type=jnp.float32
            ).astype(jnp.bfloat16)

        rdmas = []
        for i, (src, dst) in enumerate(
            [(c_loc, c_nbr), (wuk_ref, wuk_nbr), (wuv_ref, wuv_nbr)]
        ):
            r = pltpu.make_async_remote_copy(
                src_ref=src, dst_ref=dst,
                send_sem=send_sems.at[i], recv_sem=recv_sems.at[i],
                device_id=nbr, device_id_type=pl.DeviceIdType.MESH,
            )
            r.start()
            rdmas.append(r)

        for b in range(B):
            xv = x_ref[b]
            q_buf[pl.ds(b * S, S), :] = jnp.dot(
                xv, wq_ref[:], preferred_element_type=jnp.float32
            ).astype(jnp.bfloat16)
            qr_buf[pl.ds(b * S, S), :] = jnp.dot(
                xv, wqr_ref[:], preferred_element_type=jnp.float32
            ).astype(jnp.bfloat16)
            kr_buf[pl.ds(b * S, S), :] = jnp.dot(
                xv, wkr_ref[:], preferred_element_type=jnp.float32
            ).astype(jnp.bfloat16)

        for r in rdmas:
            r.wait()

        for b in range(B):
            cl = c_loc[pl.ds(b * S, S), :]
            cn = c_nbr[pl.ds(b * S, S), :]
            k_buf[pl.ds(b * S, S), :] = (
                jnp.dot(cl, wuk_ref[:], preferred_element_type=jnp.float32)
                + jnp.dot(cn, wuk_nbr[:], preferred_element_type=jnp.float32)
            ).astype(jnp.bfloat16)
            v_buf[pl.ds(b * S, S), :] = (
                jnp.dot(cl, wuv_ref[:], preferred_element_type=jnp.float32)
                + jnp.dot(cn, wuv_nbr[:], preferred_element_type=jnp.float32)
            ).astype(jnp.bfloat16)

        for b in range(B):
            kr_b = kr_buf[pl.ds(b * S, S), :]
            qr_b = qr_buf[pl.ds(b * S, S), :]
            for h in range(H):
                q = q_buf[pl.ds(b * S, S), pl.ds(h * Dh, Dh)]
                k = k_buf[pl.ds(b * S, S), pl.ds(h * Dh, Dh)]
                v = v_buf[pl.ds(b * S, S), pl.ds(h * Dh, Dh)]
                qr = qr_b[:, h * Dr:(h + 1) * Dr]
                q160 = jnp.concatenate([q, qr], axis=1)
                k160 = jnp.concatenate([k, kr_b], axis=1)
                s = lax.dot_general(
                    q160, k160, (((1,), (1,)), ((), ())),
                    preferred_element_type=jnp.float32,
                ) * SCALE
                m = jnp.max(s, axis=1, keepdims=True)
                p = jnp.exp(s - m)
                p = p / jnp.sum(p, axis=1, keepdims=True)
                o = jnp.dot(p.astype(jnp.bfloat16), v,
                            preferred_element_type=jnp.float32)
                o_buf[pl.ds(b * S, S), pl.ds(h * Dh, Dh)] = o.astype(
                    jnp.bfloat16)

        for b in range(B):
            out_ref[b] = jnp.dot(
                o_buf[pl.ds(b * S, S), :], wo_ref[:],
                preferred_element_type=jnp.float32,
            )

    args = [a.astype(jnp.bfloat16)
            for a in (x, Wdkv, Wuk, Wuv, Wq, Wqr, Wkr, Wo)]
    return pl.pallas_call(
        body,
        out_shape=jax.ShapeDtypeStruct((B, S, D), jnp.float32),
        in_specs=[pl.BlockSpec(memory_space=pltpu.VMEM)] * 8,
        out_specs=pl.BlockSpec(memory_space=pltpu.VMEM),
        scratch_shapes=[
            pltpu.VMEM((BS, DC_SH), jnp.bfloat16),
            pltpu.VMEM((BS, DC_SH), jnp.bfloat16),
            pltpu.VMEM((DC_SH, D), jnp.bfloat16),
            pltpu.VMEM((DC_SH, D), jnp.bfloat16),
            pltpu.VMEM((BS, D), jnp.bfloat16),
            pltpu.VMEM((BS, H * Dr), jnp.bfloat16),
            pltpu.VMEM((BS, Dr), jnp.bfloat16),
            pltpu.VMEM((BS, D), jnp.bfloat16),
            pltpu.VMEM((BS, D), jnp.bfloat16),
            pltpu.VMEM((BS, D), jnp.bfloat16),
            pltpu.SemaphoreType.DMA((3,)),
            pltpu.SemaphoreType.DMA((3,)),
        ],
        compiler_params=pltpu.CompilerParams(collective_id=0),
    )(*args)


# baseline (device time: 86240 ns/iter reference)
import jax
import jax.numpy as jnp
from jax import lax
from jax.experimental import pallas as pl
from jax.experimental.pallas import tpu as pltpu

B, S, D, DC, H, Dh, Dr = 2, 512, 2048, 256, 16, 128, 32
DC_SH = DC // 2
BS = B * S
RC = 256
SCALE = (Dh + Dr) ** -0.5


def kernel(x, Wdkv, Wuk, Wuv, Wq, Wqr, Wkr, Wo):
    def body(x_ref, wdkv_ref, wuk_ref, wuv_ref, wq_ref, wqr_ref, wkr_ref,
             wo_ref, out_ref,
             c_loc, c_nbr, wuk_nbr, wuv_nbr,
             q_buf, qr_buf, kr_buf, k_buf, v_buf,
             send_sems, recv_sems):
        my_x = lax.axis_index("x")
        my_y = lax.axis_index("y")
        nbr = (1 - my_x, my_y)

        barrier = pltpu.get_barrier_semaphore()
        pl.semaphore_signal(barrier, inc=1, device_id=nbr,
                            device_id_type=pl.DeviceIdType.MESH)
        pl.semaphore_wait(barrier, 1)

        for b in range(B):
            xvb = x_ref[b]
            for r in range(0, S, RC):
                c_loc[pl.ds(b * S + r, RC), :] = jnp.dot(
                    xvb[r:r + RC], wdkv_ref[:],
                    preferred_element_type=jnp.float32,
                ).astype(jnp.bfloat16)

        rdmas = []
        for i, (src, dst) in enumerate(
            [(c_loc, c_nbr), (wuk_ref, wuk_nbr), (wuv_ref, wuv_nbr)]
        ):
            r = pltpu.make_async_remote_copy(
                src_ref=src, dst_ref=dst,
                send_sem=send_sems.at[i], recv_sem=recv_sems.at[i],
                device_id=nbr, device_id_type=pl.DeviceIdType.MESH,
            )
            r.start()
            rdmas.append(r)

        for b in range(B):
            xvb = x_ref[b]
            for r in range(0, S, RC):
                xc = xvb[r:r + RC]
                q_buf[pl.ds(b * S + r, RC), :] = jnp.dot(
                    xc, wq_ref[:], preferred_element_type=jnp.float32
                ).astype(jnp.bfloat16)
                qr_buf[pl.ds(b * S + r, RC), :] = jnp.dot(
                    xc, wqr_ref[:], preferred_element_type=jnp.float32
                ).astype(jnp.bfloat16)
                kr_buf[pl.ds(b * S + r, RC), :] = jnp.dot(
                    xc, wkr_ref[:], preferred_element_type=jnp.float32
                ).astype(jnp.bfloat16)

        for r in rdmas:
            r.wait()

        for r in range(0, BS, RC):
            cl = c_loc[pl.ds(r, RC), :]
            cn = c_nbr[pl.ds(r, RC), :]
            k_buf[pl.ds(r, RC), :] = (
                jnp.dot(cl, wuk_ref[:], preferred_element_type=jnp.float32)
                + jnp.dot(cn, wuk_nbr[:], preferred_element_type=jnp.float32)
            ).astype(jnp.bfloat16)
            v_buf[pl.ds(r, RC), :] = (
                jnp.dot(cl, wuv_ref[:], preferred_element_type=jnp.float32)
                + jnp.dot(cn, wuv_nbr[:], preferred_element_type=jnp.float32)
            ).astype(jnp.bfloat16)

        for b in range(B):
            kr_b = kr_buf[pl.ds(b * S, S), :]
            qr_b = qr_buf[pl.ds(b * S, S), :]
            for h in range(H):
                q = q_buf[pl.ds(b * S, S), pl.ds(h * Dh, Dh)]
                k = k_buf[pl.ds(b * S, S), pl.ds(h * Dh, Dh)]
                v = v_buf[pl.ds(b * S, S), pl.ds(h * Dh, Dh)]
                qr = qr_b[:, h * Dr:(h + 1) * Dr]
                q160 = jnp.concatenate([q, qr], axis=1)
                k160 = jnp.concatenate([k, kr_b], axis=1)
                s = lax.dot_general(
                    q160, k160, (((1,), (1,)), ((), ())),
                    preferred_element_type=jnp.float32,
                ) * SCALE
                m = jnp.max(s, axis=1, keepdims=True)
                p = jnp.exp(s - m)
                p = p / jnp.sum(p, axis=1, keepdims=True)
                o = jnp.dot(p.astype(jnp.bfloat16), v,
                            preferred_element_type=jnp.float32)
                q_buf[pl.ds(b * S, S), pl.ds(h * Dh, Dh)] = o.astype(
                    jnp.bfloat16)

        for b in range(B):
            for r in range(0, S, RC):
                out_ref[b, pl.ds(r, RC), :] = jnp.dot(
                    q_buf[pl.ds(b * S + r, RC), :], wo_ref[:],
                    preferred_element_type=jnp.float32,
                )

    args = [a.astype(jnp.bfloat16)
            for a in (x, Wdkv, Wuk, Wuv, Wq, Wqr, Wkr, Wo)]
    return pl.pallas_call(
        body,
        out_shape=jax.ShapeDtypeStruct((B, S, D), jnp.float32),
        in_specs=[pl.BlockSpec(memory_space=pltpu.VMEM)] * 8,
        out_specs=pl.BlockSpec(memory_space=pltpu.VMEM),
        scratch_shapes=[
            pltpu.VMEM((BS, DC_SH), jnp.bfloat16),
            pltpu.VMEM((BS, DC_SH), jnp.bfloat16),
            pltpu.VMEM((DC_SH, D), jnp.bfloat16),
            pltpu.VMEM((DC_SH, D), jnp.bfloat16),
            pltpu.VMEM((BS, D), jnp.bfloat16),
            pltpu.VMEM((BS, H * Dr), jnp.bfloat16),
            pltpu.VMEM((BS, Dr), jnp.bfloat16),
            pltpu.VMEM((BS, D), jnp.bfloat16),
            pltpu.VMEM((BS, D), jnp.bfloat16),
            pltpu.SemaphoreType.DMA((3,)),
            pltpu.SemaphoreType.DMA((3,)),
        ],
        compiler_params=pltpu.CompilerParams(
            collective_id=0,
            vmem_limit_bytes=34 * 1024 * 1024,
        ),
    )(*args)


# device time: 77003 ns/iter; 1.1200x vs baseline; 1.1200x over previous
import jax
import jax.numpy as jnp
from jax import lax
from jax.experimental import pallas as pl
from jax.experimental.pallas import tpu as pltpu

B, S, D, DC, H, Dh, Dr = 2, 512, 2048, 256, 16, 128, 32
DC_SH = DC // 2
HL = H // 2
HD = HL * Dh
BS = B * S
RC = 256
SCALE = (Dh + Dr) ** -0.5


def kernel(x, Wdkv, Wuk, Wuv, Wq, Wqr, Wkr, Wo):
    def body(x_ref, wdkv_ref, wuk_ref, wuv_ref, wq_ref, wqr_ref, wkr_ref,
             wo_my_ref, wo_ot_ref, out_ref,
             c_loc, c_nbr, wuk_nbr, wuv_nbr,
             q_buf, qr_buf, kr_buf, k_buf, v_buf, o_nbr,
             send_sems, recv_sems, o_send_sems, o_recv_sems):
        my_x = lax.axis_index("x")
        my_y = lax.axis_index("y")
        x_nbr = (1 - my_x, my_y)
        y_nbr = (my_x, 1 - my_y)

        barrier = pltpu.get_barrier_semaphore()
        for nbr in (x_nbr, y_nbr):
            pl.semaphore_signal(barrier, inc=1, device_id=nbr,
                                device_id_type=pl.DeviceIdType.MESH)
        pl.semaphore_wait(barrier, 2)

        for b in range(B):
            xvb = x_ref[b]
            for r in range(0, S, RC):
                c_loc[pl.ds(b * S + r, RC), :] = jnp.dot(
                    xvb[r:r + RC], wdkv_ref[:],
                    preferred_element_type=jnp.float32,
                ).astype(jnp.bfloat16)

        rdmas = []
        for i, (src, dst) in enumerate(
            [(c_loc, c_nbr), (wuk_ref, wuk_nbr), (wuv_ref, wuv_nbr)]
        ):
            r = pltpu.make_async_remote_copy(
                src_ref=src, dst_ref=dst,
                send_sem=send_sems.at[i], recv_sem=recv_sems.at[i],
                device_id=x_nbr, device_id_type=pl.DeviceIdType.MESH,
            )
            r.start()
            rdmas.append(r)

        for b in range(B):
            xvb = x_ref[b]
            for r in range(0, S, RC):
                xc = xvb[r:r + RC]
                q_buf[pl.ds(b * S + r, RC), :] = jnp.dot(
                    xc, wq_ref[:], preferred_element_type=jnp.float32
                ).astype(jnp.bfloat16)
                qr_buf[pl.ds(b * S + r, RC), :] = jnp.dot(
                    xc, wqr_ref[:], preferred_element_type=jnp.float32
                ).astype(jnp.bfloat16)
                kr_buf[pl.ds(b * S + r, RC), :] = jnp.dot(
                    xc, wkr_ref[:], preferred_element_type=jnp.float32
                ).astype(jnp.bfloat16)

        for r in rdmas:
            r.wait()

        for r in range(0, BS, RC):
            cl = c_loc[pl.ds(r, RC), :]
            cn = c_nbr[pl.ds(r, RC), :]
            k_buf[pl.ds(r, RC), :] = (
                jnp.dot(cl, wuk_ref[:], preferred_element_type=jnp.float32)
                + jnp.dot(cn, wuk_nbr[:], preferred_element_type=jnp.float32)
            ).astype(jnp.bfloat16)
            v_buf[pl.ds(r, RC), :] = (
                jnp.dot(cl, wuv_ref[:], preferred_element_type=jnp.float32)
                + jnp.dot(cn, wuv_nbr[:], preferred_element_type=jnp.float32)
            ).astype(jnp.bfloat16)

        o_rdmas = []
        for h in range(HL):
            for b in range(B):
                kr_b = kr_buf[pl.ds(b * S, S), :]
                qr_b = qr_buf[pl.ds(b * S, S), :]
                q = q_buf[pl.ds(b * S, S), pl.ds(h * Dh, Dh)]
                k = k_buf[pl.ds(b * S, S), pl.ds(h * Dh, Dh)]
                v = v_buf[pl.ds(b * S, S), pl.ds(h * Dh, Dh)]
                qr = qr_b[:, h * Dr:(h + 1) * Dr]
                q160 = jnp.concatenate([q, qr], axis=1)
                k160 = jnp.concatenate([k, kr_b], axis=1)
                s = lax.dot_general(
                    q160, k160, (((1,), (1,)), ((), ())),
                    preferred_element_type=jnp.float32,
                ) * SCALE
                m = jnp.max(s, axis=1, keepdims=True)
                p = jnp.exp(s - m)
                p = p / jnp.sum(p, axis=1, keepdims=True)
                o = jnp.dot(p.astype(jnp.bfloat16), v,
                            preferred_element_type=jnp.float32)
                q_buf[pl.ds(b * S, S), pl.ds(h * Dh, Dh)] = o.astype(
                    jnp.bfloat16)
            r = pltpu.make_async_remote_copy(
                src_ref=q_buf.at[:, pl.ds(h * Dh, Dh)],
                dst_ref=o_nbr.at[:, pl.ds(h * Dh, Dh)],
                send_sem=o_send_sems.at[h], recv_sem=o_recv_sems.at[h],
                device_id=y_nbr, device_id_type=pl.DeviceIdType.MESH,
            )
            r.start()
            o_rdmas.append(r)

        for b in range(B):
            for r in range(0, S, RC):
                out_ref[b, pl.ds(r, RC), :] = jnp.dot(
                    q_buf[pl.ds(b * S + r, RC), :], wo_my_ref[:],
                    preferred_element_type=jnp.float32,
                )

        for r in o_rdmas:
            r.wait_recv()

        for b in range(B):
            for r in range(0, S, RC):
                out_ref[b, pl.ds(r, RC), :] += jnp.dot(
                    o_nbr[pl.ds(b * S + r, RC), :], wo_ot_ref[:],
                    preferred_element_type=jnp.float32,
                )

        for r in o_rdmas:
            r.wait_send()

    y = lax.axis_index("y")
    wq_h = lax.dynamic_slice_in_dim(Wq, y * HD, HD, 1)
    wqr_h = lax.dynamic_slice_in_dim(Wqr, y * HL * Dr, HL * Dr, 1)
    wuk_h = lax.dynamic_slice_in_dim(Wuk, y * HD, HD, 1)
    wuv_h = lax.dynamic_slice_in_dim(Wuv, y * HD, HD, 1)
    wo_my = lax.dynamic_slice_in_dim(Wo, y * HD, HD, 0)
    wo_ot = lax.dynamic_slice_in_dim(Wo, (1 - y) * HD, HD, 0)

    args = [a.astype(jnp.bfloat16)
            for a in (x, Wdkv, wuk_h, wuv_h, wq_h, wqr_h, Wkr, wo_my, wo_ot)]
    return pl.pallas_call(
        body,
        out_shape=jax.ShapeDtypeStruct((B, S, D), jnp.float32),
        in_specs=[pl.BlockSpec(memory_space=pltpu.VMEM)] * 9,
        out_specs=pl.BlockSpec(memory_space=pltpu.VMEM),
        scratch_shapes=[
            pltpu.VMEM((BS, DC_SH), jnp.bfloat16),
            pltpu.VMEM((BS, DC_SH), jnp.bfloat16),
            pltpu.VMEM((DC_SH, HD), jnp.bfloat16),
            pltpu.VMEM((DC_SH, HD), jnp.bfloat16),
            pltpu.VMEM((BS, HD), jnp.bfloat16),
            pltpu.VMEM((BS, HL * Dr), jnp.bfloat16),
            pltpu.VMEM((BS, Dr), jnp.bfloat16),
            pltpu.VMEM((BS, HD), jnp.bfloat16),
            pltpu.VMEM((BS, HD), jnp.bfloat16),
            pltpu.VMEM((BS, HD), jnp.bfloat16),
            pltpu.SemaphoreType.DMA((3,)),
            pltpu.SemaphoreType.DMA((3,)),
            pltpu.SemaphoreType.DMA((HL,)),
            pltpu.SemaphoreType.DMA((HL,)),
        ],
        compiler_params=pltpu.CompilerParams(
            collective_id=0,
            vmem_limit_bytes=34 * 1024 * 1024,
        ),
    )(*args)


# device time: 69823 ns/iter; 1.2351x vs baseline; 1.1028x over previous
import jax
import jax.numpy as jnp
from jax import lax
from jax.experimental import pallas as pl
from jax.experimental.pallas import tpu as pltpu

B, S, D, DC, H, Dh, Dr = 2, 512, 2048, 256, 16, 128, 32
DC_SH = DC // 2
HL = H // 2
HD = HL * Dh
BS = B * S
RC = 256
WQ_C = 256
WO_C = 256
SCALE = (Dh + Dr) ** -0.5
BF = jnp.bfloat16


def kernel(x, Wdkv, Wuk, Wuv, Wq, Wqr, Wkr, Wo):
    def body(x_ref, wdkv_ref, wuk_ref, wuv_ref, wq_hbm, wqr_ref, wkr_ref,
             wo_hbm, out_ref,
             xb, wdkv_b, wuk_my, wuv_my, wq_b, wqr_b, wkr_b,
             stg_q, stg_o,
             c_loc, c_nbr, wuk_nbr, wuv_nbr,
             q_buf, qr_buf, kr_buf, k_buf, v_buf, o_nbr,
             send_sems, recv_sems, o_send_sems, o_recv_sems,
             stgq_sems, stgo_sems):
        my_x = lax.axis_index("x")
        my_y = lax.axis_index("y")
        x_nbr = (1 - my_x, my_y)
        y_nbr = (my_x, 1 - my_y)

        barrier = pltpu.get_barrier_semaphore()
        for nbr in (x_nbr, y_nbr):
            pl.semaphore_signal(barrier, inc=1, device_id=nbr,
                                device_id_type=pl.DeviceIdType.MESH)
        pl.semaphore_wait(barrier, 2)

        n_wq = D // WQ_C
        wq_dmas = [
            pltpu.make_async_copy(
                wq_hbm.at[pl.ds(i * WQ_C, WQ_C), pl.ds(my_y * HD, HD)],
                stg_q.at[i % 2], stgq_sems.at[i % 2])
            for i in range(n_wq)
        ]
        wq_dmas[0].start()
        wq_dmas[1].start()

        for b in range(B):
            for r in range(0, S, RC):
                xb[pl.ds(b * S + r, RC), :] = x_ref[b][r:r + RC].astype(BF)
        wdkv_b[:] = wdkv_ref[:].astype(BF)
        wuk_my[:] = wuk_ref[:, pl.ds(my_y * HD, HD)].astype(BF)
        wuv_my[:] = wuv_ref[:, pl.ds(my_y * HD, HD)].astype(BF)
        wqr_b[:] = wqr_ref[:, pl.ds(my_y * HL * Dr, HL * Dr)].astype(BF)
        wkr_b[:] = wkr_ref[:].astype(BF)

        for r in range(0, BS, RC):
            c_loc[pl.ds(r, RC), :] = jnp.dot(
                xb[pl.ds(r, RC), :], wdkv_b[:],
                preferred_element_type=jnp.float32,
            ).astype(BF)

        rdmas = []
        for i, (src, dst) in enumerate(
            [(c_loc, c_nbr), (wuk_my, wuk_nbr), (wuv_my, wuv_nbr)]
        ):
            r = pltpu.make_async_remote_copy(
                src_ref=src, dst_ref=dst,
                send_sem=send_sems.at[i], recv_sem=recv_sems.at[i],
                device_id=x_nbr, device_id_type=pl.DeviceIdType.MESH,
            )
            r.start()
            rdmas.append(r)

        for i in range(n_wq):
            wq_dmas[i].wait()
            if i + 2 < n_wq:
                wq_dmas[i + 2].start()
            wq_b[pl.ds(i * WQ_C, WQ_C), :] = stg_q[i % 2].astype(BF)

        for r in range(0, BS, RC):
            xc = xb[pl.ds(r, RC), :]
            q_buf[pl.ds(r, RC), :] = jnp.dot(
                xc, wq_b[:], preferred_element_type=jnp.float32
            ).astype(BF)
            qr_buf[pl.ds(r, RC), :] = jnp.dot(
                xc, wqr_b[:], preferred_element_type=jnp.float32
            ).astype(BF)
            kr_buf[pl.ds(r, RC), :] = jnp.dot(
                xc, wkr_b[:], preferred_element_type=jnp.float32
            ).astype(BF)

        for r in rdmas:
            r.wait()

        for r in range(0, BS, RC):
            cl = c_loc[pl.ds(r, RC), :]
            cn = c_nbr[pl.ds(r, RC), :]
            k_buf[pl.ds(r, RC), :] = (
                jnp.dot(cl, wuk_my[:], preferred_element_type=jnp.float32)
                + jnp.dot(cn, wuk_nbr[:], preferred_element_type=jnp.float32)
            ).astype(BF)
            v_buf[pl.ds(r, RC), :] = (
                jnp.dot(cl, wuv_my[:], preferred_element_type=jnp.float32)
                + jnp.dot(cn, wuv_nbr[:], preferred_element_type=jnp.float32)
            ).astype(BF)

        o_rdmas = []
        for h in range(HL):
            for b in range(B):
                kr_s = kr_buf[pl.ds(b * S, S), :]
                qr_s = qr_buf[pl.ds(b * S, S), :]
                q = q_buf[pl.ds(b * S, S), pl.ds(h * Dh, Dh)]
                k = k_buf[pl.ds(b * S, S), pl.ds(h * Dh, Dh)]
                v = v_buf[pl.ds(b * S, S), pl.ds(h * Dh, Dh)]
                qr = qr_s[:, h * Dr:(h + 1) * Dr]
                q160 = jnp.concatenate([q, qr], axis=1)
                k160 = jnp.concatenate([k, kr_s], axis=1)
                s = lax.dot_general(
                    q160, k160, (((1,), (1,)), ((), ())),
                    preferred_element_type=jnp.float32,
                ) * SCALE
                m = jnp.max(s, axis=1, keepdims=True)
                p = jnp.exp(s - m)
                p = p / jnp.sum(p, axis=1, keepdims=True)
                o = jnp.dot(p.astype(BF), v,
                            preferred_element_type=jnp.float32)
                q_buf[pl.ds(b * S, S), pl.ds(h * Dh, Dh)] = o.astype(BF)
            r = pltpu.make_async_remote_copy(
                src_ref=q_buf.at[:, pl.ds(h * Dh, Dh)],
                dst_ref=o_nbr.at[:, pl.ds(h * Dh, Dh)],
                send_sem=o_send_sems.at[h], recv_sem=o_recv_sems.at[h],
                device_id=y_nbr, device_id_type=pl.DeviceIdType.MESH,
            )
            r.start()
            o_rdmas.append(r)

        n_wo = HD // WO_C

        def wo_dma(stage, i):
            row0 = (my_y if stage == 0 else 1 - my_y) * HD + i * WO_C
            return pltpu.make_async_copy(
                wo_hbm.at[pl.ds(row0, WO_C), :],
                stg_o.at[i % 2], stgo_sems.at[i % 2])

        dmas_a = [wo_dma(0, i) for i in range(n_wo)]
        dmas_b = [wo_dma(1, i) for i in range(n_wo)]
        dmas_a[0].start()
        dmas_a[1].start()
        for i in range(n_wo):
            dmas_a[i].wait()
            wo_c = stg_o[i % 2].astype(BF)
            for r in range(0, BS, RC):
                ob = q_buf[pl.ds(r, RC), pl.ds(i * WO_C, WO_C)]
                part = jnp.dot(ob, wo_c, preferred_element_type=jnp.float32)
                b, rr = r // S, r % S
                if i == 0:
                    out_ref[b, pl.ds(rr, RC), :] = part
                else:
                    out_ref[b, pl.ds(rr, RC), :] += part
            if i + 2 < n_wo:
                dmas_a[i + 2].start()
            elif i + 2 < 2 * n_wo:
                dmas_b[i + 2 - n_wo].start()

        for r in o_rdmas:
            r.wait_recv()

        for i in range(n_wo):
            dmas_b[i].wait()
            wo_c = stg_o[i % 2].astype(BF)
            for r in range(0, BS, RC):
                ob = o_nbr[pl.ds(r, RC), pl.ds(i * WO_C, WO_C)]
                part = jnp.dot(ob, wo_c, preferred_element_type=jnp.float32)
                b, rr = r // S, r % S
                out_ref[b, pl.ds(rr, RC), :] += part
            if i + 2 < n_wo:
                dmas_b[i + 2].start()

        for r in o_rdmas:
            r.wait_send()

    return pl.pallas_call(
        body,
        out_shape=jax.ShapeDtypeStruct((B, S, D), jnp.float32),
        in_specs=[
            pl.BlockSpec(memory_space=pltpu.VMEM),
            pl.BlockSpec(memory_space=pltpu.VMEM),
            pl.BlockSpec(memory_space=pltpu.VMEM),
            pl.BlockSpec(memory_space=pltpu.VMEM),
            pl.BlockSpec(memory_space=pltpu.MemorySpace.HBM),
            pl.BlockSpec(memory_space=pltpu.VMEM),
            pl.BlockSpec(memory_space=pltpu.VMEM),
            pl.BlockSpec(memory_space=pltpu.MemorySpace.HBM),
        ],
        out_specs=pl.BlockSpec(memory_space=pltpu.VMEM),
        scratch_shapes=[
            pltpu.VMEM((BS, D), BF),
            pltpu.VMEM((D, DC_SH), BF),
            pltpu.VMEM((DC_SH, HD), BF),
            pltpu.VMEM((DC_SH, HD), BF),
            pltpu.VMEM((D, HD), BF),
            pltpu.VMEM((D, HL * Dr), BF),
            pltpu.VMEM((D, Dr), BF),
            pltpu.VMEM((2, WQ_C, HD), jnp.float32),
            pltpu.VMEM((2, WO_C, D), jnp.float32),
            pltpu.VMEM((BS, DC_SH), BF),
            pltpu.VMEM((BS, DC_SH), BF),
            pltpu.VMEM((DC_SH, HD), BF),
            pltpu.VMEM((DC_SH, HD), BF),
            pltpu.VMEM((BS, HD), BF),
            pltpu.VMEM((BS, HL * Dr), BF),
            pltpu.VMEM((BS, Dr), BF),
            pltpu.VMEM((BS, HD), BF),
            pltpu.VMEM((BS, HD), BF),
            pltpu.VMEM((BS, HD), BF),
            pltpu.SemaphoreType.DMA((3,)),
            pltpu.SemaphoreType.DMA((3,)),
            pltpu.SemaphoreType.DMA((HL,)),
            pltpu.SemaphoreType.DMA((HL,)),
            pltpu.SemaphoreType.DMA((2,)),
            pltpu.SemaphoreType.DMA((2,)),
        ],
        compiler_params=pltpu.CompilerParams(
            collective_id=0,
            vmem_limit_bytes=54 * 1024 * 1024,
        ),
    )(x, Wdkv, Wuk, Wuv, Wq, Wqr, Wkr, Wo)


# device time: 68463 ns/iter; 1.2597x vs baseline; 1.0199x over previous
import jax
import jax.numpy as jnp
from jax import lax
from jax.experimental import pallas as pl
from jax.experimental.pallas import tpu as pltpu

B, S, D, DC, H, Dh, Dr = 2, 512, 2048, 256, 16, 128, 32
DC_SH = DC // 2
HL = H // 2
HD = HL * Dh
BS = B * S
RC = 256
WQ_C = 256
WO_C = 256
SCALE = (Dh + Dr) ** -0.5
BF = jnp.bfloat16


def kernel(x, Wdkv, Wuk, Wuv, Wq, Wqr, Wkr, Wo):
    def body(x_ref, wdkv_ref, wuk_ref, wuv_ref, wq_hbm, wqr_ref, wkr_ref,
             wo_hbm, out_ref,
             xb, wdkv_b, wuk_my, wuv_my, wq_b, wqr_b, wkr_b,
             stg_q, stg_o,
             c_loc, c_nbr, wuk_nbr, wuv_nbr,
             q_buf, qr_buf, kr_buf, k_buf, v_buf, o_nbr,
             send_sems, recv_sems, o_send_sems, o_recv_sems,
             stgq_sems, stgo_sems):
        my_x = lax.axis_index("x")
        my_y = lax.axis_index("y")
        x_nbr = (1 - my_x, my_y)
        y_nbr = (my_x, 1 - my_y)

        n_wq = D // WQ_C
        wq_dmas = [
            pltpu.make_async_copy(
                wq_hbm.at[pl.ds(i * WQ_C, WQ_C), pl.ds(my_y * HD, HD)],
                stg_q.at[i % 2], stgq_sems.at[i % 2])
            for i in range(n_wq)
        ]
        wq_dmas[0].start()
        wq_dmas[1].start()

        barrier = pltpu.get_barrier_semaphore()
        for nbr in (x_nbr, y_nbr):
            pl.semaphore_signal(barrier, inc=1, device_id=nbr,
                                device_id_type=pl.DeviceIdType.MESH)
        pl.semaphore_wait(barrier, 2)

        for b in range(B):
            for r in range(0, S, RC):
                xb[pl.ds(b * S + r, RC), :] = x_ref[b][r:r + RC].astype(BF)
        wdkv_b[:] = wdkv_ref[:].astype(BF)
        wuk_my[:] = wuk_ref[:, pl.ds(my_y * HD, HD)].astype(BF)
        wuv_my[:] = wuv_ref[:, pl.ds(my_y * HD, HD)].astype(BF)
        wqr_b[:] = wqr_ref[:, pl.ds(my_y * HL * Dr, HL * Dr)].astype(BF)
        wkr_b[:] = wkr_ref[:].astype(BF)

        for r in range(0, BS, RC):
            c_loc[pl.ds(r, RC), :] = jnp.dot(
                xb[pl.ds(r, RC), :], wdkv_b[:],
                preferred_element_type=jnp.float32,
            ).astype(BF)

        rdmas = []
        for i, (src, dst) in enumerate(
            [(c_loc, c_nbr), (wuk_my, wuk_nbr), (wuv_my, wuv_nbr)]
        ):
            r = pltpu.make_async_remote_copy(
                src_ref=src, dst_ref=dst,
                send_sem=send_sems.at[i], recv_sem=recv_sems.at[i],
                device_id=x_nbr, device_id_type=pl.DeviceIdType.MESH,
            )
            r.start()
            rdmas.append(r)

        for i in range(n_wq):
            wq_dmas[i].wait()
            if i + 2 < n_wq:
                wq_dmas[i + 2].start()
            wq_b[pl.ds(i * WQ_C, WQ_C), :] = stg_q[i % 2].astype(BF)

        for r in range(0, BS, RC):
            xc = xb[pl.ds(r, RC), :]
            q_buf[pl.ds(r, RC), :] = (jnp.dot(
                xc, wq_b[:], preferred_element_type=jnp.float32
            ) * SCALE).astype(BF)
            qr_buf[pl.ds(r, RC), :] = (jnp.dot(
                xc, wqr_b[:], preferred_element_type=jnp.float32
            ) * SCALE).astype(BF)
            kr_buf[pl.ds(r, RC), :] = jnp.dot(
                xc, wkr_b[:], preferred_element_type=jnp.float32
            ).astype(BF)

        for r in rdmas:
            r.wait()

        for r in range(0, BS, RC):
            cl = c_loc[pl.ds(r, RC), :]
            cn = c_nbr[pl.ds(r, RC), :]
            k_buf[pl.ds(r, RC), :] = (
                jnp.dot(cl, wuk_my[:], preferred_element_type=jnp.float32)
                + jnp.dot(cn, wuk_nbr[:], preferred_element_type=jnp.float32)
            ).astype(BF)
            v_buf[pl.ds(r, RC), :] = (
                jnp.dot(cl, wuv_my[:], preferred_element_type=jnp.float32)
                + jnp.dot(cn, wuv_nbr[:], preferred_element_type=jnp.float32)
            ).astype(BF)

        n_wo = HD // WO_C

        def wo_dma(stage, i):
            row0 = (my_y if stage == 0 else 1 - my_y) * HD + i * WO_C
            return pltpu.make_async_copy(
                wo_hbm.at[pl.ds(row0, WO_C), :],
                stg_o.at[i % 2], stgo_sems.at[i % 2])

        dmas_a = [wo_dma(0, i) for i in range(n_wo)]
        dmas_b = [wo_dma(1, i) for i in range(n_wo)]
        dmas_a[0].start()
        dmas_a[1].start()

        o_rdmas = []
        for h in range(HL):
            for b in range(B):
                kr_s = kr_buf[pl.ds(b * S, S), :]
                qr_s = qr_buf[pl.ds(b * S, S), :]
                q = q_buf[pl.ds(b * S, S), pl.ds(h * Dh, Dh)]
                k = k_buf[pl.ds(b * S, S), pl.ds(h * Dh, Dh)]
                v = v_buf[pl.ds(b * S, S), pl.ds(h * Dh, Dh)]
                qr = qr_s[:, h * Dr:(h + 1) * Dr]
                q160 = jnp.concatenate([q, qr], axis=1)
                k160 = jnp.concatenate([k, kr_s], axis=1)
                s = lax.dot_general(
                    q160, k160, (((1,), (1,)), ((), ())),
                    preferred_element_type=jnp.float32,
                )
                p = jnp.exp(s)
                denom = jnp.sum(p, axis=1, keepdims=True)
                o = jnp.dot(p.astype(BF), v,
                            preferred_element_type=jnp.float32)
                o = o * (1.0 / denom)
                q_buf[pl.ds(b * S, S), pl.ds(h * Dh, Dh)] = o.astype(BF)
            r = pltpu.make_async_remote_copy(
                src_ref=q_buf.at[:, pl.ds(h * Dh, Dh)],
                dst_ref=o_nbr.at[:, pl.ds(h * Dh, Dh)],
                send_sem=o_send_sems.at[h], recv_sem=o_recv_sems.at[h],
                device_id=y_nbr, device_id_type=pl.DeviceIdType.MESH,
            )
            r.start()
            o_rdmas.append(r)

        for i in range(n_wo):
            dmas_a[i].wait()
            wo_c = stg_o[i % 2].astype(BF)
            for r in range(0, BS, RC):
                ob = q_buf[pl.ds(r, RC), pl.ds(i * WO_C, WO_C)]
                part = jnp.dot(ob, wo_c, preferred_element_type=jnp.float32)
                b, rr = r // S, r % S
                if i == 0:
                    out_ref[b, pl.ds(rr, RC), :] = part
                else:
                    out_ref[b, pl.ds(rr, RC), :] += part
            if i + 2 < n_wo:
                dmas_a[i + 2].start()
            elif i + 2 < 2 * n_wo:
                dmas_b[i + 2 - n_wo].start()

        for r in o_rdmas:
            r.wait_recv()

        for i in range(n_wo):
            dmas_b[i].wait()
            wo_c = stg_o[i % 2].astype(BF)
            for r in range(0, BS, RC):
                ob = o_nbr[pl.ds(r, RC), pl.ds(i * WO_C, WO_C)]
                part = jnp.dot(ob, wo_c, preferred_element_type=jnp.float32)
                b, rr = r // S, r % S
                out_ref[b, pl.ds(rr, RC), :] += part
            if i + 2 < n_wo:
                dmas_b[i + 2].start()

        for r in o_rdmas:
            r.wait_send()

    return pl.pallas_call(
        body,
        out_shape=jax.ShapeDtypeStruct((B, S, D), jnp.float32),
        in_specs=[
            pl.BlockSpec(memory_space=pltpu.VMEM),
            pl.BlockSpec(memory_space=pltpu.VMEM),
            pl.BlockSpec(memory_space=pltpu.VMEM),
            pl.BlockSpec(memory_space=pltpu.VMEM),
            pl.BlockSpec(memory_space=pltpu.MemorySpace.HBM),
            pl.BlockSpec(memory_space=pltpu.VMEM),
            pl.BlockSpec(memory_space=pltpu.VMEM),
            pl.BlockSpec(memory_space=pltpu.MemorySpace.HBM),
        ],
        out_specs=pl.BlockSpec(memory_space=pltpu.VMEM),
        scratch_shapes=[
            pltpu.VMEM((BS, D), BF),
            pltpu.VMEM((D, DC_SH), BF),
            pltpu.VMEM((DC_SH, HD), BF),
            pltpu.VMEM((DC_SH, HD), BF),
            pltpu.VMEM((D, HD), BF),
            pltpu.VMEM((D, HL * Dr), BF),
            pltpu.VMEM((D, Dr), BF),
            pltpu.VMEM((2, WQ_C, HD), jnp.float32),
            pltpu.VMEM((2, WO_C, D), jnp.float32),
            pltpu.VMEM((BS, DC_SH), BF),
            pltpu.VMEM((BS, DC_SH), BF),
            pltpu.VMEM((DC_SH, HD), BF),
            pltpu.VMEM((DC_SH, HD), BF),
            pltpu.VMEM((BS, HD), BF),
            pltpu.VMEM((BS, HL * Dr), BF),
            pltpu.VMEM((BS, Dr), BF),
            pltpu.VMEM((BS, HD), BF),
            pltpu.VMEM((BS, HD), BF),
            pltpu.VMEM((BS, HD), BF),
            pltpu.SemaphoreType.DMA((3,)),
            pltpu.SemaphoreType.DMA((3,)),
            pltpu.SemaphoreType.DMA((HL,)),
            pltpu.SemaphoreType.DMA((HL,)),
            pltpu.SemaphoreType.DMA((2,)),
            pltpu.SemaphoreType.DMA((2,)),
        ],
        compiler_params=pltpu.CompilerParams(
            collective_id=0,
            vmem_limit_bytes=54 * 1024 * 1024,
        ),
    )(x, Wdkv, Wuk, Wuv, Wq, Wqr, Wkr, Wo)


# device time: 62358 ns/iter; 1.3830x vs baseline; 1.0979x over previous
import jax
import jax.numpy as jnp
from jax import lax
from jax.experimental import pallas as pl
from jax.experimental.pallas import tpu as pltpu

B, S, D, DC, H, Dh, Dr = 2, 512, 2048, 256, 16, 128, 32
DC_SH = DC // 2
HL = H // 2
HD = HL * Dh
BS = B * S
RC = 256
WQ_C = 256
WO_C = 256
SCALE = (Dh + Dr) ** -0.5
BF = jnp.bfloat16


def kernel(x, Wdkv, Wuk, Wuv, Wq, Wqr, Wkr, Wo):
    def body(x_ref, wdkv_ref, wuk_ref, wuv_ref, wq_hbm, wqr_ref, wkr_ref,
             wo_hbm, out_ref,
             xb, wdkv_b, wuk_my, wuv_my, wq_b, wqr_b, wkr_b,
             stg_q, stg_o,
             c_loc, c_nbr, wuk_nbr, wuv_nbr,
             q_buf, qr_buf, kr_buf, k_buf, v_buf, o_nbr,
             send_sems, recv_sems, o_send_sems, o_recv_sems,
             stgq_sems, stgo_sems):
        my_x = lax.axis_index("x")
        my_y = lax.axis_index("y")
        x_nbr = (1 - my_x, my_y)
        y_nbr = (my_x, 1 - my_y)

        n_wq = D // WQ_C
        wq_dmas = [
            pltpu.make_async_copy(
                wq_hbm.at[pl.ds(i * WQ_C, WQ_C), pl.ds(my_y * HD, HD)],
                stg_q.at[i % 2], stgq_sems.at[i % 2])
            for i in range(n_wq)
        ]
        wq_dmas[0].start()
        wq_dmas[1].start()

        barrier = pltpu.get_barrier_semaphore()
        for nbr in (x_nbr, y_nbr):
            pl.semaphore_signal(barrier, inc=1, device_id=nbr,
                                device_id_type=pl.DeviceIdType.MESH)
        pl.semaphore_wait(barrier, 2)

        for b in range(B):
            for r in range(0, S, RC):
                xb[pl.ds(b * S + r, RC), :] = x_ref[b][r:r + RC].astype(BF)
        wdkv_b[:] = wdkv_ref[:].astype(BF)
        wuk_my[:] = wuk_ref[:, pl.ds(my_y * HD, HD)].astype(BF)
        wuv_my[:] = wuv_ref[:, pl.ds(my_y * HD, HD)].astype(BF)
        wqr_b[:] = wqr_ref[:, pl.ds(my_y * HL * Dr, HL * Dr)].astype(BF)
        wkr_b[:] = wkr_ref[:].astype(BF)

        for r in range(0, BS, RC):
            c_loc[pl.ds(r, RC), :] = jnp.dot(
                xb[pl.ds(r, RC), :], wdkv_b[:],
                preferred_element_type=jnp.float32,
            ).astype(BF)

        rdmas = []
        for i, (src, dst) in enumerate(
            [(c_loc, c_nbr), (wuk_my, wuk_nbr), (wuv_my, wuv_nbr)]
        ):
            r = pltpu.make_async_remote_copy(
                src_ref=src, dst_ref=dst,
                send_sem=send_sems.at[i], recv_sem=recv_sems.at[i],
                device_id=x_nbr, device_id_type=pl.DeviceIdType.MESH,
            )
            r.start()
            rdmas.append(r)

        for i in range(n_wq):
            wq_dmas[i].wait()
            if i + 2 < n_wq:
                wq_dmas[i + 2].start()
            wq_b[pl.ds(i * WQ_C, WQ_C), :] = stg_q[i % 2].astype(BF)

        for r in range(0, BS, RC):
            cl = c_loc[pl.ds(r, RC), :]
            k_buf[pl.ds(r, RC), :] = jnp.dot(
                cl, wuk_my[:], preferred_element_type=jnp.float32
            ).astype(BF)
            v_buf[pl.ds(r, RC), :] = jnp.dot(
                cl, wuv_my[:], preferred_element_type=jnp.float32
            ).astype(BF)

        for r in range(0, BS, RC):
            xc = xb[pl.ds(r, RC), :]
            q_buf[pl.ds(r, RC), :] = (jnp.dot(
                xc, wq_b[:], preferred_element_type=jnp.float32
            ) * SCALE).astype(BF)
            qr_buf[pl.ds(r, RC), :] = (jnp.dot(
                xc, wqr_b[:], preferred_element_type=jnp.float32
            ) * SCALE).astype(BF)
            kr_buf[pl.ds(r, RC), :] = jnp.dot(
                xc, wkr_b[:], preferred_element_type=jnp.float32
            ).astype(BF)

        for r in rdmas:
            r.wait()

        for r in range(0, BS, RC):
            cn = c_nbr[pl.ds(r, RC), :]
            k_buf[pl.ds(r, RC), :] = (
                k_buf[pl.ds(r, RC), :].astype(jnp.float32)
                + jnp.dot(cn, wuk_nbr[:], preferred_element_type=jnp.float32)
            ).astype(BF)
            v_buf[pl.ds(r, RC), :] = (
                v_buf[pl.ds(r, RC), :].astype(jnp.float32)
                + jnp.dot(cn, wuv_nbr[:], preferred_element_type=jnp.float32)
            ).astype(BF)

        n_wo = HD // WO_C

        def wo_dma(stage, i):
            row0 = (my_y if stage == 0 else 1 - my_y) * HD + i * WO_C
            return pltpu.make_async_copy(
                wo_hbm.at[pl.ds(row0, WO_C), :],
                stg_o.at[i % 2], stgo_sems.at[i % 2])

        dmas_a = [wo_dma(0, i) for i in range(n_wo)]
        dmas_b = [wo_dma(1, i) for i in range(n_wo)]
        dmas_a[0].start()
        dmas_a[1].start()

        o_rdmas = []
        for h in range(HL):
            for b in range(B):
                kr_s = kr_buf[pl.ds(b * S, S), :]
                qr_s = qr_buf[pl.ds(b * S, S), :]
                q = q_buf[pl.ds(b * S, S), pl.ds(h * Dh, Dh)]
                k = k_buf[pl.ds(b * S, S), pl.ds(h * Dh, Dh)]
                v = v_buf[pl.ds(b * S, S), pl.ds(h * Dh, Dh)]
                qr = qr_s[:, h * Dr:(h + 1) * Dr]
                q160 = jnp.concatenate([q, qr], axis=1)
                k160 = jnp.concatenate([k, kr_s], axis=1)
                s = lax.dot_general(
                    q160, k160, (((1,), (1,)), ((), ())),
                    preferred_element_type=jnp.float32,
                )
                p = jnp.exp(s)
                denom = jnp.sum(p, axis=1, keepdims=True)
                o = jnp.dot(p.astype(BF), v,
                            preferred_element_type=jnp.float32)
                o = o * (1.0 / denom)
                q_buf[pl.ds(b * S, S), pl.ds(h * Dh, Dh)] = o.astype(BF)
            r = pltpu.make_async_remote_copy(
                src_ref=q_buf.at[:, pl.ds(h * Dh, Dh)],
                dst_ref=o_nbr.at[:, pl.ds(h * Dh, Dh)],
                send_sem=o_send_sems.at[h], recv_sem=o_recv_sems.at[h],
                device_id=y_nbr, device_id_type=pl.DeviceIdType.MESH,
            )
            r.start()
            o_rdmas.append(r)

        for i in range(n_wo):
            dmas_a[i].wait()
            wo_c = stg_o[i % 2].astype(BF)
            for r in range(0, BS, RC):
                ob = q_buf[pl.ds(r, RC), pl.ds(i * WO_C, WO_C)]
                part = jnp.dot(ob, wo_c, preferred_element_type=jnp.float32)
                b, rr = r // S, r % S
                if i == 0:
                    out_ref[b, pl.ds(rr, RC), :] = part
                else:
                    out_ref[b, pl.ds(rr, RC), :] += part
            if i + 2 < n_wo:
                dmas_a[i + 2].start()
            elif i + 2 < 2 * n_wo:
                dmas_b[i + 2 - n_wo].start()

        for i in range(n_wo):
            dmas_b[i].wait()
            wo_c = stg_o[i % 2].astype(BF)
            o_rdmas[2 * i].wait_recv()
            o_rdmas[2 * i + 1].wait_recv()
            for r in range(0, BS, RC):
                ob = o_nbr[pl.ds(r, RC), pl.ds(i * WO_C, WO_C)]
                part = jnp.dot(ob, wo_c, preferred_element_type=jnp.float32)
                b, rr = r // S, r % S
                out_ref[b, pl.ds(rr, RC), :] += part
            if i + 2 < n_wo:
                dmas_b[i + 2].start()

        for r in o_rdmas:
            r.wait_send()

    return pl.pallas_call(
        body,
        out_shape=jax.ShapeDtypeStruct((B, S, D), jnp.float32),
        in_specs=[
            pl.BlockSpec(memory_space=pltpu.VMEM),
            pl.BlockSpec(memory_space=pltpu.VMEM),
            pl.BlockSpec(memory_space=pltpu.VMEM),
            pl.BlockSpec(memory_space=pltpu.VMEM),
            pl.BlockSpec(memory_space=pltpu.MemorySpace.HBM),
            pl.BlockSpec(memory_space=pltpu.VMEM),
            pl.BlockSpec(memory_space=pltpu.VMEM),
            pl.BlockSpec(memory_space=pltpu.MemorySpace.HBM),
        ],
        out_specs=pl.BlockSpec(memory_space=pltpu.VMEM),
        scratch_shapes=[
            pltpu.VMEM((BS, D), BF),
            pltpu.VMEM((D, DC_SH), BF),
            pltpu.VMEM((DC_SH, HD), BF),
            pltpu.VMEM((DC_SH, HD), BF),
            pltpu.VMEM((D, HD), BF),
            pltpu.VMEM((D, HL * Dr), BF),
            pltpu.VMEM((D, Dr), BF),
            pltpu.VMEM((2, WQ_C, HD), jnp.float32),
            pltpu.VMEM((2, WO_C, D), jnp.float32),
            pltpu.VMEM((BS, DC_SH), BF),
            pltpu.VMEM((BS, DC_SH), BF),
            pltpu.VMEM((DC_SH, HD), BF),
            pltpu.VMEM((DC_SH, HD), BF),
            pltpu.VMEM((BS, HD), BF),
            pltpu.VMEM((BS, HL * Dr), BF),
            pltpu.VMEM((BS, Dr), BF),
            pltpu.VMEM((BS, HD), BF),
            pltpu.VMEM((BS, HD), BF),
            pltpu.VMEM((BS, HD), BF),
            pltpu.SemaphoreType.DMA((3,)),
            pltpu.SemaphoreType.DMA((3,)),
            pltpu.SemaphoreType.DMA((HL,)),
            pltpu.SemaphoreType.DMA((HL,)),
            pltpu.SemaphoreType.DMA((2,)),
            pltpu.SemaphoreType.DMA((2,)),
        ],
        compiler_params=pltpu.CompilerParams(
            collective_id=0,
            vmem_limit_bytes=54 * 1024 * 1024,
        ),
    )(x, Wdkv, Wuk, Wuv, Wq, Wqr, Wkr, Wo)
